# Initial kernel scaffold; baseline (speedup 1.0000x reference)
import jax
import jax.numpy as jnp
from jax.experimental import pallas as pl


def kernel(x, edge_index, W_gcn, b_gcn, W_lin, b_lin):
    raise NotImplementedError("write your pallas kernel here")


# speedup vs baseline: 6.8012x; 6.8012x over previous
import jax
import jax.numpy as jnp
from jax.experimental import pallas as pl
from jax.experimental.pallas import tpu as pltpu


def _fused_body(ei_ref, xf_ref, wg_ref, bg_ref, wl_ref, bl_ref, out_ref):
    row = ei_ref[0:1, :]
    col = ei_ref[1:2, :]
    iot = jax.lax.broadcasted_iota(jnp.int32, (32, 64), 0)
    e_src = (iot == row).astype(jnp.float32)
    e_dst = (iot == col).astype(jnp.float32)
    deg = 1.0 + jnp.sum(e_dst, axis=1, keepdims=True)
    dinv = jax.lax.rsqrt(deg)
    adj = jax.lax.dot_general(
        e_dst, e_src, (((1,), (1,)), ((), ())),
        preferred_element_type=jnp.float32)
    eye = (jax.lax.broadcasted_iota(jnp.int32, (32, 32), 0)
           == jax.lax.broadcasted_iota(jnp.int32, (32, 32), 1)).astype(jnp.float32)
    s_mat = dinv * jnp.transpose(dinv) * (adj + eye)
    agg = jnp.dot(s_mat, xf_ref[...], preferred_element_type=jnp.float32)
    hs = [jax.nn.relu(agg * wg_ref[0, k] + bg_ref[k]) for k in range(5)]
    h = jnp.concatenate(hs, axis=1)
    logits = jnp.dot(h, wl_ref[...], preferred_element_type=jnp.float32) + bl_ref[...]
    m = jnp.max(logits, axis=1, keepdims=True)
    e = jnp.exp(logits - m)
    out_ref[...] = e / jnp.sum(e, axis=1, keepdims=True)


def kernel(x, edge_index, W_gcn, b_gcn, W_lin, b_lin):
    xf = x.reshape(32, 160)
    ei = edge_index.astype(jnp.int32)
    wl_t = jnp.transpose(W_lin.reshape(3, 10, 16, 5), (3, 2, 1, 0)).reshape(800, 3)
    return pl.pallas_call(
        _fused_body,
        out_shape=jax.ShapeDtypeStruct((32, 3), jnp.float32),
        in_specs=[
            pl.BlockSpec(memory_space=pltpu.VMEM),
            pl.BlockSpec(memory_space=pltpu.VMEM),
            pl.BlockSpec(memory_space=pltpu.SMEM),
            pl.BlockSpec(memory_space=pltpu.SMEM),
            pl.BlockSpec(memory_space=pltpu.VMEM),
            pl.BlockSpec(memory_space=pltpu.VMEM),
        ],
    )(ei, xf, W_gcn, b_gcn, wl_t, b_lin.reshape(1, 3))
